# baseline (speedup 1.0000x reference)
import jax
import jax.numpy as jnp
from jax.experimental import pallas as pl
from jax.experimental.pallas import tpu as pltpu

_B = 32
_N = 127
_T = 10
_FEAT = 128
_AGG = 32
_RT = _B * _N
_RC = _RT // 2
_AROWS = 7 * _RC
_BROWS = 112
_MAIN = 2016
_TAIL = _RC - _MAIN

_M = [2 ** (6 - t) for t in range(7)]
_LO = [m - 1 for m in _M]
_NW = [_B * m for m in _M]
_OFF = [0]
for _n in _NW:
    _OFF.append(_OFF[-1] + _n)

_POST = []
_stack = [(0, False)]
while _stack:
    _nd, _done = _stack.pop()
    if _done:
        _POST.append(_nd)
    else:
        _stack.append((_nd, True))
        if 2 * _nd + 2 < _N:
            _stack.append((2 * _nd + 2, False))
        if 2 * _nd + 1 < _N:
            _stack.append((2 * _nd + 1, False))


def _node_tk(nd):
    lvl = (nd + 1).bit_length() - 1
    t = 6 - lvl
    return t, nd - _LO[t]


def _tree_body(emb_ref, tail_ref, we_ref, be_ref, wc_ref, bc_ref, wsm_ref,
               bsm_ref, out_ref, pmax_ref, nat_ref, res_ref, w_ref, b_ref,
               sem_ref):
    i = pl.program_id(0)

    copies = [
        pltpu.make_async_copy(we_ref, w_ref.at[0:_FEAT, :], sem_ref.at[0]),
        pltpu.make_async_copy(wc_ref, w_ref.at[_FEAT:2 * _FEAT, :],
                              sem_ref.at[1]),
        pltpu.make_async_copy(wsm_ref, w_ref.at[2 * _FEAT:3 * _FEAT, :],
                              sem_ref.at[2]),
        pltpu.make_async_copy(be_ref, b_ref.at[0:1, :], sem_ref.at[3]),
        pltpu.make_async_copy(bc_ref, b_ref.at[1:2, :], sem_ref.at[4]),
        pltpu.make_async_copy(bsm_ref, b_ref.at[2:3, :], sem_ref.at[5]),
    ]

    @pl.when(i == 0)
    def _start():
        for c in copies:
            c.start()

    pooled = emb_ref[pl.ds(1, _MAIN, 7), :]
    for j in range(1, _T):
        pooled = pooled + emb_ref[pl.ds(13 * j + 1, _MAIN, 7), :]
    tacc = None
    for j in range(_T):
        a_len = -(-(7 * _RC - 1 - 13 * j) // 7) - _MAIN
        parts = []
        if a_len > 0:
            parts.append(emb_ref[pl.ds(13 * j + 1 + 7 * _MAIN, a_len, 7), :])
        if a_len < _TAIL:
            boff = 13 * j + 1 + 7 * (_MAIN + a_len) - _AROWS
            parts.append(tail_ref[pl.ds(boff, _TAIL - a_len, 7), :])
        tj = parts[0] if len(parts) == 1 else jnp.concatenate(parts, axis=0)
        tacc = tj if tacc is None else tacc + tj
    pooled = jnp.concatenate([pooled, tacc], axis=0) * (1.0 / _T)

    @pl.when(i == 0)
    def _wait():
        for c in copies:
            c.wait()

    enc = jnp.maximum(
        jnp.dot(pooled, w_ref[0:_FEAT, :],
                preferred_element_type=jnp.float32) + b_ref[0:1, :], 0.0)
    nat_ref[pl.ds(i * _RC, _RC), :] = (
        jnp.dot(enc, w_ref[_FEAT:2 * _FEAT, :],
                preferred_element_type=jnp.float32) + b_ref[1:2, :])

    @pl.when(i == 1)
    def _tree():
        for t in range(7):
            lo, m, o = _LO[t], _M[t], _OFF[t]
            for k in range(m):
                res_ref[o + _B * k:o + _B * (k + 1), :] = (
                    nat_ref[pl.ds(lo + k, _B, _N), :])

        ws = w_ref[2 * _FEAT:3 * _FEAT, :]
        bs2 = 2.0 * b_ref[2:3, :]
        for t in range(1, 7):
            o, n, po = _OFF[t], _NW[t], _OFF[t - 1]
            v = res_ref[po:po + 2 * n, :].reshape(n // _B, 2, _B, _FEAT)
            csum = (v[:, 0] + v[:, 1]).reshape(n, _FEAT)
            res_ref[o:o + n, :] = (
                res_ref[o:o + n, :]
                + jnp.dot(csum, ws, preferred_element_type=jnp.float32) + bs2)

        acc = None
        for idx, nd in enumerate(_POST):
            t, k = _node_tk(nd)
            slab = jnp.maximum(
                res_ref[_OFF[t] + _B * k:_OFF[t] + _B * (k + 1), :], 0.0)
            out_ref[idx, :, :] = slab[:, :_AGG]
            acc = slab if acc is None else jnp.maximum(acc, slab)
        pmax_ref[...] = acc[:, :_AGG]


def kernel(emb_table, w_enc_pad, b_enc_pad, w_c_pad, b_c_pad, w_sum_pad,
           b_sum_pad):
    hbm = pl.BlockSpec(memory_space=pl.ANY)
    node_stack, pooled_out = pl.pallas_call(
        _tree_body,
        grid=(2,),
        out_shape=(jax.ShapeDtypeStruct((_N, _B, _AGG), jnp.float32),
                   jax.ShapeDtypeStruct((_B, _AGG), jnp.float32)),
        in_specs=[
            pl.BlockSpec((_AROWS, _FEAT), lambda i: (i, 0)),
            pl.BlockSpec((_BROWS, _FEAT), lambda i: (127 * (i + 1), 0)),
            hbm, hbm, hbm, hbm, hbm, hbm,
        ],
        out_specs=(pl.BlockSpec((_N, _B, _AGG), lambda i: (0, 0, 0)),
                   pl.BlockSpec((_B, _AGG), lambda i: (0, 0))),
        scratch_shapes=[pltpu.VMEM((_RT, _FEAT), jnp.float32),
                        pltpu.VMEM((_RT, _FEAT), jnp.float32),
                        pltpu.VMEM((3 * _FEAT, _FEAT), jnp.float32),
                        pltpu.VMEM((3, _FEAT), jnp.float32),
                        pltpu.SemaphoreType.DMA((6,))],
        compiler_params=pltpu.CompilerParams(
            dimension_semantics=("arbitrary",),
            vmem_limit_bytes=48 << 20),
    )(emb_table, emb_table, w_enc_pad, b_enc_pad, w_c_pad, b_c_pad,
      w_sum_pad, b_sum_pad)
    return node_stack, pooled_out


# speedup vs baseline: 1.3167x; 1.3167x over previous
import jax
import jax.numpy as jnp
from jax.experimental import pallas as pl
from jax.experimental.pallas import tpu as pltpu

_B = 32
_N = 127
_T = 10
_FEAT = 128
_AGG = 32
_RT = _B * _N
_RC = _RT // 2
_AROWS = 7 * _RC
_BROWS = 112
_MAIN = 2016
_TAIL = _RC - _MAIN

_M = [2 ** (6 - t) for t in range(7)]
_LO = [m - 1 for m in _M]
_NW = [_B * m for m in _M]
_OFF = [0]
for _n in _NW:
    _OFF.append(_OFF[-1] + _n)

_POST = []
_stack = [(0, False)]
while _stack:
    _nd, _done = _stack.pop()
    if _done:
        _POST.append(_nd)
    else:
        _stack.append((_nd, True))
        if 2 * _nd + 2 < _N:
            _stack.append((2 * _nd + 2, False))
        if 2 * _nd + 1 < _N:
            _stack.append((2 * _nd + 1, False))


def _node_tk(nd):
    lvl = (nd + 1).bit_length() - 1
    t = 6 - lvl
    return t, nd - _LO[t]


def _tree_body(emb_ref, tail_ref, w_ref, b_ref, out_ref, pmax_ref,
               nat_ref, res_ref):
    i = pl.program_id(0)

    pooled = emb_ref[pl.ds(1, _MAIN, 7), :]
    for j in range(1, _T):
        pooled = pooled + emb_ref[pl.ds(13 * j + 1, _MAIN, 7), :]
    tacc = None
    for j in range(_T):
        a_len = -(-(7 * _RC - 1 - 13 * j) // 7) - _MAIN
        parts = []
        if a_len > 0:
            parts.append(emb_ref[pl.ds(13 * j + 1 + 7 * _MAIN, a_len, 7), :])
        if a_len < _TAIL:
            boff = 13 * j + 1 + 7 * (_MAIN + a_len) - _AROWS
            parts.append(tail_ref[pl.ds(boff, _TAIL - a_len, 7), :])
        tj = parts[0] if len(parts) == 1 else jnp.concatenate(parts, axis=0)
        tacc = tj if tacc is None else tacc + tj
    pooled = jnp.concatenate([pooled, tacc], axis=0) * (1.0 / _T)

    enc = jnp.maximum(
        jnp.dot(pooled, w_ref[0:_FEAT, :],
                preferred_element_type=jnp.float32) + b_ref[0:1, :], 0.0)
    nat_ref[pl.ds(i * _RC, _RC), :] = (
        jnp.dot(enc, w_ref[_FEAT:2 * _FEAT, :],
                preferred_element_type=jnp.float32) + b_ref[1:2, :])

    @pl.when(i == 1)
    def _tree():
        for t in range(7):
            lo, m, o = _LO[t], _M[t], _OFF[t]
            for k in range(m):
                res_ref[o + _B * k:o + _B * (k + 1), :] = (
                    nat_ref[pl.ds(lo + k, _B, _N), :])

        ws = w_ref[2 * _FEAT:3 * _FEAT, :]
        bs2 = 2.0 * b_ref[2:3, :]
        for t in range(1, 7):
            o, n, po = _OFF[t], _NW[t], _OFF[t - 1]
            v = res_ref[po:po + 2 * n, :].reshape(n // _B, 2, _B, _FEAT)
            csum = (v[:, 0] + v[:, 1]).reshape(n, _FEAT)
            res_ref[o:o + n, :] = (
                res_ref[o:o + n, :]
                + jnp.dot(csum, ws, preferred_element_type=jnp.float32) + bs2)

        acc = None
        for idx, nd in enumerate(_POST):
            t, k = _node_tk(nd)
            slab = jnp.maximum(
                res_ref[_OFF[t] + _B * k:_OFF[t] + _B * (k + 1), :], 0.0)
            out_ref[idx, :, :] = slab[:, :_AGG]
            acc = slab if acc is None else jnp.maximum(acc, slab)
        pmax_ref[...] = acc[:, :_AGG]


def kernel(emb_table, w_enc_pad, b_enc_pad, w_c_pad, b_c_pad, w_sum_pad,
           b_sum_pad):
    wcat = jnp.concatenate([w_enc_pad, w_c_pad, w_sum_pad], axis=0)
    bcat = jnp.concatenate([b_enc_pad, b_c_pad, b_sum_pad], axis=0)
    node_stack, pooled_out = pl.pallas_call(
        _tree_body,
        grid=(2,),
        out_shape=(jax.ShapeDtypeStruct((_N, _B, _AGG), jnp.float32),
                   jax.ShapeDtypeStruct((_B, _AGG), jnp.float32)),
        in_specs=[
            pl.BlockSpec((_AROWS, _FEAT), lambda i: (i, 0)),
            pl.BlockSpec((_BROWS, _FEAT), lambda i: (127 * (i + 1), 0)),
            pl.BlockSpec((3 * _FEAT, _FEAT), lambda i: (0, 0)),
            pl.BlockSpec((3, _FEAT), lambda i: (0, 0)),
        ],
        out_specs=(pl.BlockSpec((_N, _B, _AGG), lambda i: (0, 0, 0)),
                   pl.BlockSpec((_B, _AGG), lambda i: (0, 0))),
        scratch_shapes=[pltpu.VMEM((_RT, _FEAT), jnp.float32),
                        pltpu.VMEM((_RT, _FEAT), jnp.float32)],
        compiler_params=pltpu.CompilerParams(
            dimension_semantics=("arbitrary",),
            vmem_limit_bytes=48 << 20),
    )(emb_table, emb_table, wcat, bcat)
    return node_stack, pooled_out
